# Initial kernel scaffold; baseline (speedup 1.0000x reference)
import jax
import jax.numpy as jnp
from jax.experimental import pallas as pl


def kernel(x, bpe_table, pos_table):
    raise NotImplementedError("write your pallas kernel here")


# speedup vs baseline: 5.9941x; 5.9941x over previous
import functools

import jax
import jax.numpy as jnp
from jax import lax
from jax.experimental import pallas as pl
from jax.experimental.pallas import tpu as pltpu
from jax.experimental.pallas import tpu_sc as plsc

SEQ = 200
HID = 64
BATCH = 4096

_info = plsc.get_sparse_core_info()
NC, NS = _info.num_cores, _info.num_subcores
NW = NC * NS
B_PER_W = BATCH // NW


def _emb_body(x_hbm, bpe_hbm, pos_hbm, out_hbm, idx_v, rows_v, pos_v, sem):
    wid = lax.axis_index("s") * NC + lax.axis_index("c")
    b0 = wid * B_PER_W

    pltpu.sync_copy(pos_hbm.at[pl.ds(0, SEQ)], pos_v)

    def per_batch_row(k, _):
        b = b0 + k
        pltpu.sync_copy(x_hbm.at[b], idx_v)
        pltpu.async_copy(bpe_hbm.at[idx_v], rows_v, sem).wait()

        def add_pos(r, _):
            for j in range(HID // 16):
                sl = pl.ds(j * 16, 16)
                plsc.addupdate(rows_v.at[r, sl], pos_v[r, sl])
            return 0

        lax.fori_loop(0, SEQ, add_pos, 0, unroll=2)
        pltpu.sync_copy(rows_v, out_hbm.at[b])
        return 0

    lax.fori_loop(0, B_PER_W, per_batch_row, 0)


@jax.jit
def _emb(x, bpe_table, pos_table):
    mesh = plsc.VectorSubcoreMesh(core_axis_name="c", subcore_axis_name="s")
    f = pl.kernel(
        _emb_body,
        out_type=jax.ShapeDtypeStruct((BATCH, SEQ, HID), jnp.float32),
        mesh=mesh,
        scratch_types=[
            pltpu.VMEM((SEQ,), jnp.int32),
            pltpu.VMEM((SEQ, HID), jnp.float32),
            pltpu.VMEM((SEQ, HID), jnp.float32),
            pltpu.SemaphoreType.DMA,
        ],
        compiler_params=pltpu.CompilerParams(use_tc_tiling_on_sc=False),
    )
    return f(x, bpe_table, pos_table)


def kernel(x, bpe_table, pos_table):
    return _emb(x, bpe_table, pos_table)


# speedup vs baseline: 8.1421x; 1.3583x over previous
import jax
import jax.numpy as jnp
from jax import lax
from jax.experimental import pallas as pl
from jax.experimental.pallas import tpu as pltpu
from jax.experimental.pallas import tpu_sc as plsc

SEQ = 200
HID = 64
BATCH = 4096

_info = plsc.get_sparse_core_info()
NC, NS = _info.num_cores, _info.num_subcores
NW = NC * NS
B_PER_W = BATCH // NW
NBUF = 4
LAG = 2


def _emb_body(x_hbm, bpe_hbm, pos_hbm, out_hbm, idx_all, pos_v, *bufs):
    rows = bufs[:NBUF]
    gsem = bufs[NBUF:2 * NBUF]
    osem = bufs[2 * NBUF:3 * NBUF]

    wid = lax.axis_index("s") * NC + lax.axis_index("c")
    b0 = wid * B_PER_W

    pltpu.sync_copy(x_hbm.at[pl.ds(b0, B_PER_W)], idx_all)
    pltpu.sync_copy(pos_hbm.at[pl.ds(0, SEQ)], pos_v)

    def start_gather(slot, buf):
        pltpu.async_copy(bpe_hbm.at[idx_all.at[slot]], rows[buf], gsem[buf])

    for k in range(LAG):
        start_gather(k, k % NBUF)

    def outer(i, _):
        for b in range(NBUF):
            k = i * NBUF + b
            p = k + LAG
            bp = (b + LAG) % NBUF

            @pl.when(jnp.logical_and(p >= NBUF, p < B_PER_W))
            def _():
                pltpu.make_async_copy(rows[bp], out_hbm.at[b0], osem[bp]).wait()

            @pl.when(p < B_PER_W)
            def _():
                start_gather(p, bp)

            pltpu.make_async_copy(
                bpe_hbm.at[idx_all.at[k]], rows[b], gsem[b]).wait()

            def add_pos(r, _):
                for j in range(HID // 16):
                    sl = pl.ds(j * 16, 16)
                    plsc.addupdate(rows[b].at[r, sl], pos_v[r, sl])
                return 0

            lax.fori_loop(0, SEQ, add_pos, 0, unroll=8)
            pltpu.async_copy(rows[b], out_hbm.at[b0 + k], osem[b])
        return 0

    lax.fori_loop(0, B_PER_W // NBUF, outer, 0)

    for b in range(NBUF):
        pltpu.make_async_copy(rows[b], out_hbm.at[b0], osem[b]).wait()


@jax.jit
def _emb(x, bpe_table, pos_table):
    mesh = plsc.VectorSubcoreMesh(core_axis_name="c", subcore_axis_name="s")
    f = pl.kernel(
        _emb_body,
        out_type=jax.ShapeDtypeStruct((BATCH, SEQ, HID), jnp.float32),
        mesh=mesh,
        scratch_types=(
            [pltpu.VMEM((B_PER_W, SEQ), jnp.int32),
             pltpu.VMEM((SEQ, HID), jnp.float32)]
            + [pltpu.VMEM((SEQ, HID), jnp.float32) for _ in range(NBUF)]
            + [pltpu.SemaphoreType.DMA for _ in range(2 * NBUF)]
        ),
        compiler_params=pltpu.CompilerParams(use_tc_tiling_on_sc=False),
    )
    return f(x, bpe_table, pos_table)


def kernel(x, bpe_table, pos_table):
    return _emb(x, bpe_table, pos_table)
